# baseline (speedup 1.0000x reference)
import jax
import jax.numpy as jnp
from jax import lax
from jax.experimental import pallas as pl
from jax.experimental.pallas import tpu as pltpu
from jax.experimental.pallas import tpu_sc as plsc

H, W = 180, 320
K = 32
C = 256
THRESH = 0.1
ROWS_PER_IMG = (H * W) // 128


def _nms_topk_body(h_ref, fm_ref, scores_ref, inds_ref, bboxt_ref, kpst_ref,
                   out_ref, smem_i, smem_v, gbuf, sem):
    h = h_ref[0, 0]
    pad_r = jnp.full((4, W), -jnp.inf, jnp.float32)
    hv = jnp.concatenate([pad_r, h, pad_r], axis=0)
    rm = hv[0:H]
    for d in range(1, 9):
        rm = jnp.maximum(rm, hv[d:d + H])
    pad_c = jnp.full((H, 4), -jnp.inf, jnp.float32)
    hc = jnp.concatenate([pad_c, rm, pad_c], axis=1)
    hm = hc[:, 0:W]
    for d in range(1, 9):
        hm = jnp.maximum(hm, hc[:, d:d + W])
    keep = (hm == h) & (h > THRESH)
    s = jnp.where(keep, h, 0.0)

    PH, PW = 256, 512
    s = jnp.concatenate([s, jnp.full((PH - H, W), -1.0, jnp.float32)], axis=0)
    s = jnp.concatenate([s, jnp.full((PH, PW - W), -1.0, jnp.float32)], axis=1)
    flat = (lax.broadcasted_iota(jnp.int32, (PH, PW), 0) * W
            + lax.broadcasted_iota(jnp.int32, (PH, PW), 1))
    colk = lax.broadcasted_iota(jnp.int32, (1, K), 1)

    def fold(v, i, axis, h):
        if axis == 0:
            vt, vb, it, ib = v[:h], v[h:], i[:h], i[h:]
        else:
            vt, vb, it, ib = v[:, :h], v[:, h:], i[:, :h], i[:, h:]
        take = (vt > vb) | ((vt == vb) & (it < ib))
        return jnp.where(take, vt, vb), jnp.where(take, it, ib)

    def step(k, carry):
        s, vals, inds = carry
        v, i = s, flat
        for h in (128, 64, 32, 16, 8, 4, 2, 1):
            v, i = fold(v, i, 0, h)
        for h in (256, 128, 64, 32, 16, 8, 4, 2, 1):
            v, i = fold(v, i, 1, h)
        m = v[0, 0]
        idx = i[0, 0]
        smem_i[k] = idx
        smem_v[k] = m
        s = jnp.where(flat == idx, -1.0, s)
        vals = jnp.where(colk == k, m, vals)
        inds = jnp.where(colk == k, idx, inds)
        return s, vals, inds

    _, vals, inds = lax.fori_loop(
        0, K, step,
        (s, jnp.zeros((1, K), jnp.float32), jnp.zeros((1, K), jnp.int32)))

    xs = inds % W
    ys = inds // W
    scores_ref[0] = vals
    inds_ref[0] = inds
    kpst_ref[0] = jnp.concatenate([xs * 4, ys * 4], axis=0)
    bboxt_ref[0] = jnp.concatenate(
        [xs - 16, ys - 16, xs + 16, ys + 16], axis=0).astype(jnp.float32) * 4.0

    b = pl.program_id(0)
    lane_iota = lax.broadcasted_iota(jnp.int32, (C, 128), 1)

    def make_dma(k, slot):
        idxk = smem_i[k]
        yk = idxk // W
        xk = idxk % W
        ya = pl.multiple_of((yk // 8) * 8, 8)
        xa = pl.multiple_of((xk // 128) * 128, 128)
        return yk - ya, xk - xa, pltpu.make_async_copy(
            fm_ref.at[b, :, pl.ds(ya, 8), pl.ds(xa, 128)],
            gbuf.at[slot], sem.at[slot])

    NBUF = 16
    rems = []
    handles = []
    for k in range(NBUF):
        yr, lx, cp = make_dma(k, k % NBUF)
        cp.start()
        rems.append((yr, lx))
        handles.append(cp)
    for k in range(K):
        handles[k].wait()
        yr, lx = rems[k]
        g = gbuf[k % NBUF, :, pl.ds(yr, 1), :]
        sel = jnp.sum(jnp.where(lane_iota == lx, g[:, 0, :], 0.0), axis=1)
        out_ref[0, k, :] = sel * smem_v[k]
        if k + NBUF < K:
            yr, lx, cpn = make_dma(k + NBUF, (k + NBUF) % NBUF)
            cpn.start()
            rems.append((yr, lx))
            handles.append(cpn)


def _nms_topk(heatmap, feature_map):
    return pl.pallas_call(
        _nms_topk_body,
        grid=(4,),
        in_specs=[
            pl.BlockSpec((1, 1, H, W), lambda b: (b, 0, 0, 0)),
            pl.BlockSpec(memory_space=pltpu.HBM),
        ],
        out_specs=[
            pl.BlockSpec((1, 1, K), lambda b: (b, 0, 0)),
            pl.BlockSpec((1, 1, K), lambda b: (b, 0, 0)),
            pl.BlockSpec((1, 4, K), lambda b: (b, 0, 0)),
            pl.BlockSpec((1, 2, K), lambda b: (b, 0, 0)),
            pl.BlockSpec((1, K, C), lambda b: (b, 0, 0)),
        ],
        out_shape=[
            jax.ShapeDtypeStruct((4, 1, K), jnp.float32),
            jax.ShapeDtypeStruct((4, 1, K), jnp.int32),
            jax.ShapeDtypeStruct((4, 4, K), jnp.float32),
            jax.ShapeDtypeStruct((4, 2, K), jnp.int32),
            jax.ShapeDtypeStruct((4, K, C), jnp.float32),
        ],
        scratch_shapes=[
            pltpu.SMEM((K,), jnp.int32),
            pltpu.SMEM((K,), jnp.float32),
            pltpu.VMEM((16, C, 8, 128), jnp.float32),
            pltpu.SemaphoreType.DMA((16,)),
        ],
    )(heatmap, feature_map)


def _gather_body(fm3, idx_hbm, sc_hbm, out_hbm,
                 idx_all, sc_all, zvec0, colbuf0, out_all, sem):
    wid = lax.axis_index("s") * 2 + lax.axis_index("c")
    pltpu.sync_copy(idx_hbm, idx_all)
    pltpu.sync_copy(sc_hbm, sc_all)
    iota = lax.iota(jnp.int32, 16)
    zero = jnp.zeros((16,), jnp.int32)
    b = wid // 8
    for r in range(16):
        zvec0[pl.ds(r * 16, 16)] = b * C + r * 16 + iota
    for j in range(4):
        kp = wid * 4 + j
        kp_vec = jnp.full((16,), kp, jnp.int32)
        p_vec = plsc.load_gather(idx_all, [kp_vec])
        s_vec = plsc.load_gather(sc_all, [kp_vec])
        p = jnp.max(p_vec)
        y = p // W
        x = p % W
        xa = pl.multiple_of((x // 128) * 128, 128)
        lane = jnp.full((16,), x - xa, jnp.int32)
        cp0 = pltpu.async_copy(fm3.at[zvec0, pl.ds(y, 1), pl.ds(xa, 128)],
                               colbuf0, sem)
        cp0.wait()
        for cc in range(16):
            rvals = plsc.load_gather(colbuf0, [cc * 16 + iota, zero, lane])
            out_all[pl.ds(j * C + cc * 16, 16)] = rvals * s_vec
    pltpu.sync_copy(out_all, out_hbm.at[pl.ds(wid * (4 * C), 4 * C)])


def _gather(fm3, inds_flat, scores_flat):
    mesh = plsc.VectorSubcoreMesh(core_axis_name="c", subcore_axis_name="s")
    return pl.kernel(
        _gather_body,
        out_type=jax.ShapeDtypeStruct((4 * K * C,), jnp.float32),
        mesh=mesh,
        compiler_params=pltpu.CompilerParams(needs_layout_passes=False),
        scratch_types=[
            pltpu.VMEM((4 * K,), jnp.int32),
            pltpu.VMEM((4 * K,), jnp.float32),
            pltpu.VMEM((C,), jnp.int32),
            pltpu.VMEM((C, 1, 128), jnp.float32),
            pltpu.VMEM((4 * C,), jnp.float32),
            pltpu.SemaphoreType.DMA,
        ],
    )(fm3, inds_flat, scores_flat)


def kernel(heatmap, feature_map):
    scores3, inds3, bboxt, kpst, out = _nms_topk(heatmap, feature_map)
    topk_scores = scores3.reshape(4, K)
    bbox = jnp.transpose(bboxt, (0, 2, 1))
    kps = jnp.transpose(kpst, (0, 2, 1))
    return out, bbox, kps, topk_scores


# speedup vs baseline: 1.1476x; 1.1476x over previous
import jax
import jax.numpy as jnp
from jax import lax
from jax.experimental import pallas as pl
from jax.experimental.pallas import tpu as pltpu

H, W = 180, 320
K = 32
C = 256
THRESH = 0.1


def _nms_topk_body(h_ref, fm_ref, scores_ref, inds_ref, bboxt_ref, kpst_ref,
                   out_ref, smem_i, smem_v, gbuf, sem):
    h = h_ref[0, 0]
    pad_r = jnp.full((4, W), -jnp.inf, jnp.float32)
    hv = jnp.concatenate([pad_r, h, pad_r], axis=0)
    rm = hv[0:H]
    for d in range(1, 9):
        rm = jnp.maximum(rm, hv[d:d + H])
    pad_c = jnp.full((H, 4), -jnp.inf, jnp.float32)
    hc = jnp.concatenate([pad_c, rm, pad_c], axis=1)
    hm = hc[:, 0:W]
    for d in range(1, 9):
        hm = jnp.maximum(hm, hc[:, d:d + W])
    keep = (hm == h) & (h > THRESH)
    s = jnp.where(keep, h, 0.0)

    flat = (lax.broadcasted_iota(jnp.int32, (H, W), 0) * W
            + lax.broadcasted_iota(jnp.int32, (H, W), 1))
    colk = lax.broadcasted_iota(jnp.int32, (1, K), 1)

    def step(k, carry):
        s, vals, inds = carry
        m = jnp.max(s)
        cand = jnp.where(s == m, flat, jnp.int32(2 ** 30))
        idx = jnp.min(cand)
        smem_i[k] = idx
        smem_v[k] = m
        s = jnp.where(flat == idx, -1.0, s)
        vals = jnp.where(colk == k, m, vals)
        inds = jnp.where(colk == k, idx, inds)
        return s, vals, inds

    _, vals, inds = lax.fori_loop(
        0, K, step,
        (s, jnp.zeros((1, K), jnp.float32), jnp.zeros((1, K), jnp.int32)))

    xs = inds % W
    ys = inds // W
    scores_ref[0] = vals
    inds_ref[0] = inds
    kpst_ref[0] = jnp.concatenate([xs * 4, ys * 4], axis=0)
    bboxt_ref[0] = jnp.concatenate(
        [xs - 16, ys - 16, xs + 16, ys + 16], axis=0).astype(jnp.float32) * 4.0

    b = pl.program_id(0)
    lane_iota = lax.broadcasted_iota(jnp.int32, (C, 128), 1)

    def make_dma(k, slot):
        idxk = smem_i[k]
        yk = idxk // W
        xk = idxk % W
        ya = pl.multiple_of((yk // 8) * 8, 8)
        xa = pl.multiple_of((xk // 128) * 128, 128)
        return yk - ya, xk - xa, pltpu.make_async_copy(
            fm_ref.at[b, :, pl.ds(ya, 8), pl.ds(xa, 128)],
            gbuf.at[slot], sem.at[slot])

    NBUF = 16
    rems = []
    handles = []
    for k in range(NBUF):
        yr, lx, cp = make_dma(k, k % NBUF)
        cp.start()
        rems.append((yr, lx))
        handles.append(cp)
    for k in range(K):
        handles[k].wait()
        yr, lx = rems[k]
        g = gbuf[k % NBUF, :, pl.ds(yr, 1), :]
        sel = jnp.sum(jnp.where(lane_iota == lx, g[:, 0, :], 0.0), axis=1)
        out_ref[0, k, :] = sel * smem_v[k]
        if k + NBUF < K:
            yr, lx, cpn = make_dma(k + NBUF, (k + NBUF) % NBUF)
            cpn.start()
            rems.append((yr, lx))
            handles.append(cpn)


def _nms_topk(heatmap, feature_map):
    return pl.pallas_call(
        _nms_topk_body,
        grid=(4,),
        in_specs=[
            pl.BlockSpec((1, 1, H, W), lambda b: (b, 0, 0, 0)),
            pl.BlockSpec(memory_space=pltpu.HBM),
        ],
        out_specs=[
            pl.BlockSpec((1, 1, K), lambda b: (b, 0, 0)),
            pl.BlockSpec((1, 1, K), lambda b: (b, 0, 0)),
            pl.BlockSpec((1, 4, K), lambda b: (b, 0, 0)),
            pl.BlockSpec((1, 2, K), lambda b: (b, 0, 0)),
            pl.BlockSpec((1, K, C), lambda b: (b, 0, 0)),
        ],
        out_shape=[
            jax.ShapeDtypeStruct((4, 1, K), jnp.float32),
            jax.ShapeDtypeStruct((4, 1, K), jnp.int32),
            jax.ShapeDtypeStruct((4, 4, K), jnp.float32),
            jax.ShapeDtypeStruct((4, 2, K), jnp.int32),
            jax.ShapeDtypeStruct((4, K, C), jnp.float32),
        ],
        scratch_shapes=[
            pltpu.SMEM((K,), jnp.int32),
            pltpu.SMEM((K,), jnp.float32),
            pltpu.VMEM((16, C, 8, 128), jnp.float32),
            pltpu.SemaphoreType.DMA((16,)),
        ],
    )(heatmap, feature_map)


def kernel(heatmap, feature_map):
    scores3, inds3, bboxt, kpst, out = _nms_topk(heatmap, feature_map)
    topk_scores = scores3.reshape(4, K)
    bbox = jnp.transpose(bboxt, (0, 2, 1))
    kps = jnp.transpose(kpst, (0, 2, 1))
    return out, bbox, kps, topk_scores
